# baseline (speedup 1.0000x reference)
import math

import jax
import jax.numpy as jnp
from jax.experimental import pallas as pl
from jax.experimental.pallas import tpu as pltpu

N = 2048
SLOTS = 4
F = 256
C = 128
E = 8
MAXLEN = 512
SLAB = MAXLEN + 8
TPAD = 528
GPAD = N + MAXLEN
OPAD = N + TPAD


def _dot_t(a, w):
    return jax.lax.dot_general(a, w, (((1,), (1,)), ((), ())),
                               preferred_element_type=jnp.float32)


def _dot(a, w):
    return jnp.dot(a, w, preferred_element_type=jnp.float32)


def _gru_gates(gi, gh, h):
    r = jax.nn.sigmoid(gi[:, :C] + gh[:, :C])
    z = jax.nn.sigmoid(gi[:, C:2 * C] + gh[:, C:2 * C])
    n = jnp.tanh(gi[:, 2 * C:] + r * gh[:, 2 * C:])
    return (1.0 - z) * n + z * h


def _lower_bound(ent_ref, e):
    pos = jnp.int32(0)
    step = N
    while step >= 1:
        cand = pos + step
        take = jnp.logical_and(cand <= N, ent_ref[cand - 1] < e)
        pos = jnp.where(take, cand, pos)
        step //= 2
    return pos


def _graph_rnn_kernel(ent_ref,
                      x_ref, bias_ref,
                      Wfuse_ref, Wi0_ref, Wh0_ref, Wi1_ref, Wh1_ref, Wfc_ref,
                      out_ref,
                      gi0_ref, packed_ref, hist_ref, tok_ref,
                      whT0_ref, wiT1_ref, whT1_ref):
    bounds = [_lower_bound(ent_ref, e) for e in range(1, E)] + [jnp.int32(N)]
    starts = [jnp.int32(0)] + bounds[:-1]
    counts = [bounds[e] - starts[e] for e in range(E)]
    sal = [starts[e] // 8 for e in range(E)]
    rdel = [starts[e] - sal[e] * 8 for e in range(E)]

    whT0_ref[...] = Wh0_ref[...].T.astype(jnp.bfloat16)
    wiT1_ref[...] = Wi1_ref[...].T.astype(jnp.bfloat16)
    whT1_ref[...] = Wh1_ref[...].T.astype(jnp.bfloat16)

    a = x_ref[:, 2, :]
    v = x_ref[:, 3, :]
    dot = jnp.sum(a * v, axis=1, keepdims=True)
    na = jnp.maximum(jnp.sqrt(jnp.sum(a * a, axis=1, keepdims=True)), 1e-8)
    nb = jnp.maximum(jnp.sqrt(jnp.sum(v * v, axis=1, keepdims=True)), 1e-8)
    sim = dot / (na * nb)
    audio = x_ref[:, 0, :] * sim
    bf = bias_ref[:, :C]
    gamma = bias_ref[:, C:2 * C]
    beta = bias_ref[:, 2 * C:3 * C]
    bi0 = bias_ref[:, 3 * C:6 * C]
    bh0 = bias_ref[:, 6 * C:9 * C]
    bi1 = bias_ref[:, 9 * C:12 * C]
    bh1 = bias_ref[:, 12 * C:15 * C]
    bfc = bias_ref[:, 15 * C:15 * C + 2]
    g = (_dot_t(audio, Wfuse_ref[:, :F])
         + _dot_t(x_ref[:, 1, :], Wfuse_ref[:, F:])
         + bf)
    g = g * (gamma * (1.0 / math.sqrt(1.0 + 1e-5))) + beta
    g = jnp.maximum(g, 0.0)
    gi0_ref[:N, :] = _dot_t(g, Wi0_ref[...]) + bi0
    gi0_ref[N:, :] = jnp.zeros((GPAD - N, 3 * C), jnp.float32)

    for e in range(E):
        packed_ref[:SLAB, e, :] = gi0_ref[pl.ds(sal[e] * 8, SLAB), :]

    tmax = counts[0] + rdel[0]
    for e in range(1, E):
        tmax = jnp.maximum(tmax, counts[e] + rdel[e])

    rv = jnp.concatenate(
        [jnp.full((1, 1), rdel[e], jnp.int32) for e in range(E)], axis=0)

    def body(t, carry):
        h0, h1, y0p = carry
        keep0 = t >= rv
        xg = packed_ref[pl.ds(t, 1), :, :].reshape(E, 3 * C)
        gh0 = _dot(h0.astype(jnp.bfloat16), whT0_ref[...]) + bh0
        h0n = jnp.where(keep0, _gru_gates(xg, gh0, h0), 0.0)

        keep1 = (t - 1) >= rv
        gi1 = _dot(y0p.astype(jnp.bfloat16), wiT1_ref[...]) + bi1
        gh1 = _dot(h1.astype(jnp.bfloat16), whT1_ref[...]) + bh1
        h1n = jnp.where(keep1, _gru_gates(gi1, gh1, h1), 0.0)
        hist_ref[pl.ds(jnp.maximum(t - 1, 0), 1), :, :] = h1n.reshape(1, E, C)
        return (h0n, h1n, h0n)

    h_init = jnp.zeros((E, C), jnp.float32)
    jax.lax.fori_loop(0, tmax + 1, body, (h_init, h_init, h_init))

    rows = jax.lax.broadcasted_iota(jnp.int32, (SLAB, 1), 0)
    for e in range(E):
        m = (rows >= rdel[e]) & (rows < rdel[e] + counts[e])
        cur = tok_ref[pl.ds(sal[e] * 8, SLAB), :]
        tok_ref[pl.ds(sal[e] * 8, SLAB), :] = jnp.where(
            m, hist_ref[:SLAB, e, :], cur)

    out_ref[...] = _dot_t(tok_ref[:N, :], Wfc_ref[...]) + bfc


def _build(interpret=False):
    return pl.pallas_call(
        _graph_rnn_kernel,
        out_shape=jax.ShapeDtypeStruct((N, 2), jnp.float32),
        in_specs=(
            [pl.BlockSpec(memory_space=pltpu.SMEM)]
            + [pl.BlockSpec(memory_space=pltpu.VMEM)] * 8
        ),
        out_specs=pl.BlockSpec(memory_space=pltpu.VMEM),
        scratch_shapes=[
            pltpu.VMEM((GPAD, 3 * C), jnp.float32),
            pltpu.VMEM((TPAD, E, 3 * C), jnp.float32),
            pltpu.VMEM((TPAD, E, C), jnp.float32),
            pltpu.VMEM((OPAD, C), jnp.float32),
            pltpu.VMEM((C, 3 * C), jnp.bfloat16),
            pltpu.VMEM((C, 3 * C), jnp.bfloat16),
            pltpu.VMEM((C, 3 * C), jnp.bfloat16),
        ],
        interpret=interpret,
    )


def kernel(x, edge_index, edge_attr, y, W_fuse, b_fuse, gamma, beta,
           W_ih0, W_hh0, b_ih0, b_hh0, W_ih1, W_hh1, b_ih1, b_hh1,
           W_fc, b_fc):
    bias_all = jnp.concatenate(
        [b_fuse, gamma, beta, b_ih0, b_hh0, b_ih1, b_hh1, b_fc])[None, :]
    call = _build()
    return call(
        y[:, -1].astype(jnp.int32),
        x, bias_all,
        W_fuse, W_ih0, W_hh0, W_ih1, W_hh1, W_fc)


# speedup vs baseline: 1.0978x; 1.0978x over previous
import math

import jax
import jax.numpy as jnp
from jax.experimental import pallas as pl
from jax.experimental.pallas import tpu as pltpu

N = 2048
SLOTS = 4
F = 256
C = 128
E = 8
MAXLEN = 512
SLAB = MAXLEN + 8
TPAD = 528
GPAD = N + MAXLEN
OPAD = N + TPAD


def _dot_t(a, w):
    return jax.lax.dot_general(a, w, (((1,), (1,)), ((), ())),
                               preferred_element_type=jnp.float32)


def _dot(a, w):
    return jnp.dot(a, w, preferred_element_type=jnp.float32)


def _gru_gates(gi, gh, h):
    r = jax.nn.sigmoid(gi[:, :C] + gh[:, :C])
    z = jax.nn.sigmoid(gi[:, C:2 * C] + gh[:, C:2 * C])
    n = jnp.tanh(gi[:, 2 * C:] + r * gh[:, 2 * C:])
    return (1.0 - z) * n + z * h


def _lower_bound(ent_ref, e):
    pos = jnp.int32(0)
    step = N
    while step >= 1:
        cand = pos + step
        take = jnp.logical_and(cand <= N, ent_ref[cand - 1] < e)
        pos = jnp.where(take, cand, pos)
        step //= 2
    return pos


def _graph_rnn_kernel(ent_ref,
                      x_ref, bias_ref,
                      Wfuse_ref, Wi0_ref, Wh0_ref, Wi1_ref, Wh1_ref, Wfc_ref,
                      out_ref,
                      gi0_ref, packed_ref, hist_ref, tok_ref,
                      whT0_ref, wiT1_ref, whT1_ref):
    bounds = [_lower_bound(ent_ref, e) for e in range(1, E)] + [jnp.int32(N)]
    starts = [jnp.int32(0)] + bounds[:-1]
    counts = [bounds[e] - starts[e] for e in range(E)]
    sal = [starts[e] // 8 for e in range(E)]
    rdel = [starts[e] - sal[e] * 8 for e in range(E)]

    whT0_ref[...] = Wh0_ref[...].T.astype(jnp.bfloat16)
    wiT1_ref[...] = Wi1_ref[...].T.astype(jnp.bfloat16)
    whT1_ref[...] = Wh1_ref[...].T.astype(jnp.bfloat16)

    a = x_ref[:, 2, :]
    v = x_ref[:, 3, :]
    dot = jnp.sum(a * v, axis=1, keepdims=True)
    na = jnp.maximum(jnp.sqrt(jnp.sum(a * a, axis=1, keepdims=True)), 1e-8)
    nb = jnp.maximum(jnp.sqrt(jnp.sum(v * v, axis=1, keepdims=True)), 1e-8)
    sim = dot / (na * nb)
    audio = x_ref[:, 0, :] * sim
    bf = bias_ref[:, :C]
    gamma = bias_ref[:, C:2 * C]
    beta = bias_ref[:, 2 * C:3 * C]
    bi0 = bias_ref[:, 3 * C:6 * C]
    bh0 = bias_ref[:, 6 * C:9 * C]
    bi1 = bias_ref[:, 9 * C:12 * C]
    bh1 = bias_ref[:, 12 * C:15 * C]
    bfc = bias_ref[:, 15 * C:15 * C + 2]
    g = (_dot_t(audio, Wfuse_ref[:, :F])
         + _dot_t(x_ref[:, 1, :], Wfuse_ref[:, F:])
         + bf)
    g = g * (gamma * (1.0 / math.sqrt(1.0 + 1e-5))) + beta
    g = jnp.maximum(g, 0.0)
    gi0_ref[:N, :] = _dot_t(g, Wi0_ref[...]) + bi0
    gi0_ref[N:, :] = jnp.zeros((GPAD - N, 3 * C), jnp.float32)

    for e in range(E):
        packed_ref[:SLAB, e, :] = gi0_ref[pl.ds(sal[e] * 8, SLAB), :]

    tmax = counts[0] + rdel[0]
    for e in range(1, E):
        tmax = jnp.maximum(tmax, counts[e] + rdel[e])

    rv = jnp.concatenate(
        [jnp.full((1, 1), rdel[e], jnp.int32) for e in range(E)], axis=0)

    def body(t, carry):
        h0, h1, y0p = carry
        keep0 = t >= rv
        xg = packed_ref[pl.ds(t, 1), :, :].reshape(E, 3 * C)
        gh0 = _dot(h0.astype(jnp.bfloat16), whT0_ref[...]) + bh0
        h0n = jnp.where(keep0, _gru_gates(xg, gh0, h0), 0.0)

        keep1 = (t - 1) >= rv
        gi1 = _dot(y0p.astype(jnp.bfloat16), wiT1_ref[...]) + bi1
        gh1 = _dot(h1.astype(jnp.bfloat16), whT1_ref[...]) + bh1
        h1n = jnp.where(keep1, _gru_gates(gi1, gh1, h1), 0.0)
        hist_ref[pl.ds(jnp.maximum(t - 1, 0), 1), :, :] = h1n.reshape(1, E, C)
        return (h0n, h1n, h0n)

    def body2(i, carry):
        return body(2 * i + 1, body(2 * i, carry))

    h_init = jnp.zeros((E, C), jnp.float32)
    jax.lax.fori_loop(0, (tmax + 2) // 2, body2, (h_init, h_init, h_init))

    rows = jax.lax.broadcasted_iota(jnp.int32, (SLAB, 1), 0)
    for e in range(E):
        m = (rows >= rdel[e]) & (rows < rdel[e] + counts[e])
        cur = tok_ref[pl.ds(sal[e] * 8, SLAB), :]
        tok_ref[pl.ds(sal[e] * 8, SLAB), :] = jnp.where(
            m, hist_ref[:SLAB, e, :], cur)

    out_ref[...] = _dot_t(tok_ref[:N, :], Wfc_ref[...]) + bfc


def _build(interpret=False):
    return pl.pallas_call(
        _graph_rnn_kernel,
        out_shape=jax.ShapeDtypeStruct((N, 2), jnp.float32),
        in_specs=(
            [pl.BlockSpec(memory_space=pltpu.SMEM)]
            + [pl.BlockSpec(memory_space=pltpu.VMEM)] * 8
        ),
        out_specs=pl.BlockSpec(memory_space=pltpu.VMEM),
        scratch_shapes=[
            pltpu.VMEM((GPAD, 3 * C), jnp.float32),
            pltpu.VMEM((TPAD, E, 3 * C), jnp.float32),
            pltpu.VMEM((TPAD, E, C), jnp.float32),
            pltpu.VMEM((OPAD, C), jnp.float32),
            pltpu.VMEM((C, 3 * C), jnp.bfloat16),
            pltpu.VMEM((C, 3 * C), jnp.bfloat16),
            pltpu.VMEM((C, 3 * C), jnp.bfloat16),
        ],
        interpret=interpret,
    )


def kernel(x, edge_index, edge_attr, y, W_fuse, b_fuse, gamma, beta,
           W_ih0, W_hh0, b_ih0, b_hh0, W_ih1, W_hh1, b_ih1, b_hh1,
           W_fc, b_fc):
    bias_all = jnp.concatenate(
        [b_fuse, gamma, beta, b_ih0, b_hh0, b_ih1, b_hh1, b_fc])[None, :]
    call = _build()
    return call(
        y[:, -1].astype(jnp.int32),
        x, bias_all,
        W_fuse, W_ih0, W_hh0, W_ih1, W_hh1, W_fc)


# speedup vs baseline: 1.1560x; 1.0530x over previous
import math

import jax
import jax.numpy as jnp
from jax.experimental import pallas as pl
from jax.experimental.pallas import tpu as pltpu

N = 2048
SLOTS = 4
F = 256
C = 128
E = 8
MAXLEN = 512
SLAB = MAXLEN + 8
TPAD = 528
GPAD = N + MAXLEN
OPAD = N + TPAD


def _dot_t(a, w):
    return jax.lax.dot_general(a, w, (((1,), (1,)), ((), ())),
                               preferred_element_type=jnp.float32)


def _dot(a, w):
    return jnp.dot(a, w, preferred_element_type=jnp.float32)


def _gru_gates(gi, gh, h):
    r = jax.nn.sigmoid(gi[:, :C] + gh[:, :C])
    z = jax.nn.sigmoid(gi[:, C:2 * C] + gh[:, C:2 * C])
    n = jnp.tanh(gi[:, 2 * C:] + r * gh[:, 2 * C:])
    return (1.0 - z) * n + z * h


def _lower_bound(ent_ref, e):
    pos = jnp.int32(0)
    step = N
    while step >= 1:
        cand = pos + step
        take = jnp.logical_and(cand <= N, ent_ref[cand - 1] < e)
        pos = jnp.where(take, cand, pos)
        step //= 2
    return pos


def _graph_rnn_kernel(ent_ref,
                      x_ref, bias_ref,
                      Wfuse_ref, Wi0_ref, Wh0_ref, Wi1_ref, Wh1_ref, Wfc_ref,
                      out_ref,
                      gi0_ref, packed_ref, hist_ref, tok_ref,
                      whT0_ref, wiT1_ref, whT1_ref):
    bounds = [_lower_bound(ent_ref, e) for e in range(1, E)] + [jnp.int32(N)]
    starts = [jnp.int32(0)] + bounds[:-1]
    counts = [bounds[e] - starts[e] for e in range(E)]
    sal = [starts[e] // 8 for e in range(E)]
    rdel = [starts[e] - sal[e] * 8 for e in range(E)]

    whT0_ref[...] = Wh0_ref[...].T.astype(jnp.bfloat16)
    wiT1_ref[...] = Wi1_ref[...].T.astype(jnp.bfloat16)
    whT1_ref[...] = Wh1_ref[...].T.astype(jnp.bfloat16)

    a = x_ref[:, 2, :]
    v = x_ref[:, 3, :]
    dot = jnp.sum(a * v, axis=1, keepdims=True)
    na = jnp.maximum(jnp.sqrt(jnp.sum(a * a, axis=1, keepdims=True)), 1e-8)
    nb = jnp.maximum(jnp.sqrt(jnp.sum(v * v, axis=1, keepdims=True)), 1e-8)
    sim = dot / (na * nb)
    audio = x_ref[:, 0, :] * sim
    bf = bias_ref[:, :C]
    gamma = bias_ref[:, C:2 * C]
    beta = bias_ref[:, 2 * C:3 * C]
    bi0 = bias_ref[:, 3 * C:6 * C]
    bh0 = bias_ref[:, 6 * C:9 * C]
    bi1 = bias_ref[:, 9 * C:12 * C]
    bh1 = bias_ref[:, 12 * C:15 * C]
    bfc = bias_ref[:, 15 * C:15 * C + 2]
    g = (_dot_t(audio, Wfuse_ref[:, :F])
         + _dot_t(x_ref[:, 1, :], Wfuse_ref[:, F:])
         + bf)
    g = g * (gamma * (1.0 / math.sqrt(1.0 + 1e-5))) + beta
    g = jnp.maximum(g, 0.0)
    gi0_ref[:N, :] = _dot_t(g, Wi0_ref[...]) + bi0
    gi0_ref[N:, :] = jnp.zeros((GPAD - N, 3 * C), jnp.float32)

    for e in range(E):
        packed_ref[:SLAB, e, :] = gi0_ref[pl.ds(sal[e] * 8, SLAB), :]

    tmax = counts[0] + rdel[0]
    for e in range(1, E):
        tmax = jnp.maximum(tmax, counts[e] + rdel[e])

    rv = jnp.concatenate(
        [jnp.full((1, 1), rdel[e], jnp.int32) for e in range(E)], axis=0)

    def body(t, carry):
        h0, h1, y0p = carry
        keep0 = t >= rv
        xg = packed_ref[pl.ds(t, 1), :, :].reshape(E, 3 * C)
        gh0 = _dot(h0.astype(jnp.bfloat16), whT0_ref[...]) + bh0
        h0n = jnp.where(keep0, _gru_gates(xg, gh0, h0), 0.0)

        keep1 = (t - 1) >= rv
        gi1 = _dot(y0p.astype(jnp.bfloat16), wiT1_ref[...]) + bi1
        gh1 = _dot(h1.astype(jnp.bfloat16), whT1_ref[...]) + bh1
        h1n = jnp.where(keep1, _gru_gates(gi1, gh1, h1), 0.0)
        hist_ref[pl.ds(jnp.maximum(t - 1, 0), 1), :, :] = h1n.reshape(1, E, C)
        return (h0n, h1n, h0n)

    def body4(i, carry):
        for k in range(4):
            carry = body(4 * i + k, carry)
        return carry

    h_init = jnp.zeros((E, C), jnp.float32)
    jax.lax.fori_loop(0, (tmax + 4) // 4, body4, (h_init, h_init, h_init))

    rows = jax.lax.broadcasted_iota(jnp.int32, (SLAB, 1), 0)
    for e in range(E):
        m = (rows >= rdel[e]) & (rows < rdel[e] + counts[e])
        cur = tok_ref[pl.ds(sal[e] * 8, SLAB), :]
        tok_ref[pl.ds(sal[e] * 8, SLAB), :] = jnp.where(
            m, hist_ref[:SLAB, e, :], cur)

    out_ref[...] = _dot_t(tok_ref[:N, :], Wfc_ref[...]) + bfc


def _build(interpret=False):
    return pl.pallas_call(
        _graph_rnn_kernel,
        out_shape=jax.ShapeDtypeStruct((N, 2), jnp.float32),
        in_specs=(
            [pl.BlockSpec(memory_space=pltpu.SMEM)]
            + [pl.BlockSpec(memory_space=pltpu.VMEM)] * 8
        ),
        out_specs=pl.BlockSpec(memory_space=pltpu.VMEM),
        scratch_shapes=[
            pltpu.VMEM((GPAD, 3 * C), jnp.float32),
            pltpu.VMEM((TPAD, E, 3 * C), jnp.float32),
            pltpu.VMEM((TPAD, E, C), jnp.float32),
            pltpu.VMEM((OPAD, C), jnp.float32),
            pltpu.VMEM((C, 3 * C), jnp.bfloat16),
            pltpu.VMEM((C, 3 * C), jnp.bfloat16),
            pltpu.VMEM((C, 3 * C), jnp.bfloat16),
        ],
        interpret=interpret,
    )


def kernel(x, edge_index, edge_attr, y, W_fuse, b_fuse, gamma, beta,
           W_ih0, W_hh0, b_ih0, b_hh0, W_ih1, W_hh1, b_ih1, b_hh1,
           W_fc, b_fc):
    bias_all = jnp.concatenate(
        [b_fuse, gamma, beta, b_ih0, b_hh0, b_ih1, b_hh1, b_fc])[None, :]
    call = _build()
    return call(
        y[:, -1].astype(jnp.int32),
        x, bias_all,
        W_fuse, W_ih0, W_hh0, W_ih1, W_hh1, W_fc)
